# baseline (speedup 1.0000x reference)
import functools

import jax
import jax.numpy as jnp
from jax import lax
from jax.experimental import pallas as pl
from jax.experimental.pallas import tpu as pltpu
from jax.experimental.pallas import tpu_sc as plsc

_B, _P, _C = 64, 8732, 21
_NP = _B * _P
_NW = 32
_PER_W = _NP // _NW
_PB = 472
_NCH = _PER_W // _PB
_NG = 30

_TAB_LO = 13270
_TAB_N = 2992


def _log_table():
    idx = jnp.arange(_TAB_N, dtype=jnp.int32)
    bits = ((idx + _TAB_LO) << 16) | 0x8000
    return jnp.log(lax.bitcast_convert_type(bits, jnp.float32))


def _ln(tab_v, x):
    bits = lax.bitcast_convert_type(x, jnp.int32)
    ti = jnp.clip(lax.shift_right_logical(bits, 16) - _TAB_LO, 0, _TAB_N - 1)
    return plsc.load_gather(tab_v, [ti])


def _body(conf_hbm, conff_hbm, loc_hbm, locf_hbm, tab_hbm, out_hbm,
          conf_v, conff_v, loc_v, locf_v, tab_v, part_v):
    wid = lax.axis_index("s") * 2 + lax.axis_index("c")
    pltpu.sync_copy(tab_hbm, tab_v)
    iota = lax.iota(jnp.int32, 16)
    zero = jnp.zeros((16,), jnp.float32)
    one = jnp.ones((16,), jnp.float32)

    def chunk_body(ci, carry):
        kl_t, loc_t, cnt_t = carry
        p0 = wid * _PER_W + ci * _PB
        pltpu.sync_copy(conf_hbm.at[pl.ds(p0 * _C, _PB * _C)], conf_v)
        pltpu.sync_copy(conff_hbm.at[pl.ds(p0 * _C, _PB * _C)], conff_v)
        pltpu.sync_copy(loc_hbm.at[pl.ds(p0 * 4, _PB * 4)], loc_v)
        pltpu.sync_copy(locf_hbm.at[pl.ds(p0 * 4, _PB * 4)], locf_v)

        def group_body(g, gcarry):
            kl_g, loc_g, cnt_g = gcarry
            base = g * 16
            valid = (base + iota) < _PB
            pidx = jnp.minimum(base + iota, _PB - 1)
            pbase_c = pidx * _C
            pbase_l = pidx * 4

            bg = plsc.load_gather(conf_v, [pbase_c])
            gq0 = plsc.load_gather(conff_v, [pbase_c])
            p0v = bg + 1e-7
            q0v = gq0 + 1e-7
            klrow0 = (q0v - p0v) * (_ln(tab_v, q0v) - _ln(tab_v, p0v))

            def class_body(c, ccarry):
                klrow_c, fg_c = ccarry
                gp = plsc.load_gather(conf_v, [pbase_c + c])
                gq = plsc.load_gather(conff_v, [pbase_c + c])
                p = gp + 1e-7
                q = gq + 1e-7
                klrow_c = klrow_c + (q - p) * (_ln(tab_v, q) - _ln(tab_v, p))
                return klrow_c, jnp.maximum(fg_c, gp)

            klrow, fg = lax.fori_loop(
                1, _C, class_body,
                (klrow0, jnp.full((16,), -1e30, jnp.float32)))

            sq = zero
            for j in range(4):
                lv = plsc.load_gather(loc_v, [pbase_l + j])
                lf = plsc.load_gather(locf_v, [pbase_l + j])
                t = lv + lf if j == 0 else lv - lf
                sq = sq + t * t

            m = jnp.logical_and(fg > bg, valid)
            kl_g = kl_g + jnp.where(m, klrow, zero)
            loc_g = loc_g + jnp.where(m, sq, zero)
            cnt_g = cnt_g + jnp.where(m, one, zero)
            return kl_g, loc_g, cnt_g

        return lax.fori_loop(0, _NG, group_body, (kl_t, loc_t, cnt_t))

    kl_t, loc_t, cnt_t = lax.fori_loop(
        0, _NCH, chunk_body, (zero, zero, zero))
    part_v[pl.ds(0, 16)] = kl_t
    part_v[pl.ds(16, 16)] = loc_t
    part_v[pl.ds(32, 16)] = cnt_t
    pltpu.sync_copy(part_v, out_hbm.at[wid])


def kernel(conf, conf_flip, loc, loc_flip):
    conf2 = conf.reshape(_NP * _C)
    conff2 = conf_flip.reshape(_NP * _C)
    loc2 = loc.reshape(_NP * 4)
    locf2 = loc_flip.reshape(_NP * 4)
    tab = _log_table()

    sc_call = functools.partial(
        pl.kernel,
        out_type=jax.ShapeDtypeStruct((_NW, 48), jnp.float32),
        mesh=plsc.VectorSubcoreMesh(core_axis_name="c", subcore_axis_name="s"),
        compiler_params=pltpu.CompilerParams(needs_layout_passes=False),
        scratch_types=[
            pltpu.VMEM((_PB * _C,), jnp.float32),
            pltpu.VMEM((_PB * _C,), jnp.float32),
            pltpu.VMEM((_PB * 4,), jnp.float32),
            pltpu.VMEM((_PB * 4,), jnp.float32),
            pltpu.VMEM((_TAB_N,), jnp.float32),
            pltpu.VMEM((48,), jnp.float32),
        ],
    )(_body)
    out = sc_call(conf2, conff2, loc2, locf2, tab)

    kl_s = jnp.sum(out[:, 0:16])
    loc_s = jnp.sum(out[:, 16:32])
    cnt = jnp.maximum(jnp.sum(out[:, 32:48]), 1.0)
    return kl_s / (2.0 * cnt) + loc_s / (4.0 * cnt)


# speedup vs baseline: 1.1655x; 1.1655x over previous
import functools

import jax
import jax.numpy as jnp
from jax import lax
from jax.experimental import pallas as pl
from jax.experimental.pallas import tpu as pltpu
from jax.experimental.pallas import tpu_sc as plsc

_B, _P, _C = 64, 8732, 21
_NP = _B * _P
_NW = 32
_PER_W = _NP // _NW
_PB = 472
_NCH = _PER_W // _PB
_NG = 30

_TAB_LO = 13270
_TAB_N = 2992


def _log_table():
    idx = jnp.arange(_TAB_N, dtype=jnp.int32)
    bits = ((idx + _TAB_LO) << 16) | 0x8000
    return jnp.log(lax.bitcast_convert_type(bits, jnp.float32))


def _ln(tab_v, x):
    bits = lax.bitcast_convert_type(x, jnp.int32)
    ti = jnp.clip(lax.shift_right_logical(bits, 16) - _TAB_LO, 0, _TAB_N - 1)
    return plsc.load_gather(tab_v, [ti])


def _body(conf_hbm, conff_hbm, loc_hbm, locf_hbm, tab_hbm, out_hbm,
          conf_v, conff_v, loc_v, locf_v, tab_v, part_v):
    wid = lax.axis_index("s") * 2 + lax.axis_index("c")
    pltpu.sync_copy(tab_hbm, tab_v)
    iota = lax.iota(jnp.int32, 16)
    zero = jnp.zeros((16,), jnp.float32)
    one = jnp.ones((16,), jnp.float32)

    def chunk_body(ci, carry):
        kl_t, loc_t, cnt_t = carry
        p0 = wid * _PER_W + ci * _PB
        pltpu.sync_copy(conf_hbm.at[pl.ds(p0, _PB), :], conf_v)
        pltpu.sync_copy(conff_hbm.at[pl.ds(p0, _PB), :], conff_v)
        pltpu.sync_copy(loc_hbm.at[pl.ds(p0, _PB), :], loc_v)
        pltpu.sync_copy(locf_hbm.at[pl.ds(p0, _PB), :], locf_v)

        def group_body(g, gcarry):
            kl_g, loc_g, cnt_g = gcarry
            base = g * 16
            valid = (base + iota) < _PB
            pidx = jnp.minimum(base + iota, _PB - 1)
            czero = jnp.zeros((16,), jnp.int32)

            bg = plsc.load_gather(conf_v, [pidx, czero])
            gq0 = plsc.load_gather(conff_v, [pidx, czero])
            p0v = bg + 1e-7
            q0v = gq0 + 1e-7
            klrow0 = (q0v - p0v) * (_ln(tab_v, q0v) - _ln(tab_v, p0v))

            def class_body(c, ccarry):
                klrow_c, fg_c = ccarry
                cc = czero + c
                gp = plsc.load_gather(conf_v, [pidx, cc])
                gq = plsc.load_gather(conff_v, [pidx, cc])
                p = gp + 1e-7
                q = gq + 1e-7
                klrow_c = klrow_c + (q - p) * (_ln(tab_v, q) - _ln(tab_v, p))
                return klrow_c, jnp.maximum(fg_c, gp)

            klrow, fg = lax.fori_loop(
                1, _C, class_body,
                (klrow0, jnp.full((16,), -1e30, jnp.float32)))

            sq = zero
            for j in range(4):
                jj = czero + j
                lv = plsc.load_gather(loc_v, [pidx, jj])
                lf = plsc.load_gather(locf_v, [pidx, jj])
                t = lv + lf if j == 0 else lv - lf
                sq = sq + t * t

            m = jnp.logical_and(fg > bg, valid)
            kl_g = kl_g + jnp.where(m, klrow, zero)
            loc_g = loc_g + jnp.where(m, sq, zero)
            cnt_g = cnt_g + jnp.where(m, one, zero)
            return kl_g, loc_g, cnt_g

        return lax.fori_loop(0, _NG, group_body, (kl_t, loc_t, cnt_t))

    kl_t, loc_t, cnt_t = lax.fori_loop(
        0, _NCH, chunk_body, (zero, zero, zero))
    part_v[pl.ds(0, 16)] = kl_t
    part_v[pl.ds(16, 16)] = loc_t
    part_v[pl.ds(32, 16)] = cnt_t
    pltpu.sync_copy(part_v, out_hbm.at[wid])


def kernel(conf, conf_flip, loc, loc_flip):
    conf2 = conf.reshape(_NP, _C)
    conff2 = conf_flip.reshape(_NP, _C)
    loc2 = loc.reshape(_NP, 4)
    locf2 = loc_flip.reshape(_NP, 4)
    tab = _log_table()

    sc_call = functools.partial(
        pl.kernel,
        out_type=jax.ShapeDtypeStruct((_NW, 48), jnp.float32),
        mesh=plsc.VectorSubcoreMesh(core_axis_name="c", subcore_axis_name="s"),
        compiler_params=pltpu.CompilerParams(
            needs_layout_passes=False, use_tc_tiling_on_sc=False),
        scratch_types=[
            pltpu.VMEM((_PB, _C), jnp.float32),
            pltpu.VMEM((_PB, _C), jnp.float32),
            pltpu.VMEM((_PB, 4), jnp.float32),
            pltpu.VMEM((_PB, 4), jnp.float32),
            pltpu.VMEM((_TAB_N,), jnp.float32),
            pltpu.VMEM((48,), jnp.float32),
        ],
    )(_body)
    out = sc_call(conf2, conff2, loc2, locf2, tab)

    kl_s = jnp.sum(out[:, 0:16])
    loc_s = jnp.sum(out[:, 16:32])
    cnt = jnp.maximum(jnp.sum(out[:, 32:48]), 1.0)
    return kl_s / (2.0 * cnt) + loc_s / (4.0 * cnt)
